# baseline (speedup 1.0000x reference)
import functools
import jax
import jax.numpy as jnp
from jax import lax
from jax.experimental import pallas as pl
from jax.experimental.pallas import tpu as pltpu
from jax.experimental.pallas import tpu_sc as plsc

D_MODEL = 64
NUM_WORKERS = 32
BLK = 128


def _make_gather(seq: int, batch: int, vpad: int):
    assert batch // BLK == NUM_WORKERS
    flat_len = D_MODEL * vpad
    mesh = plsc.VectorSubcoreMesh(core_axis_name="c", subcore_axis_name="s")

    @functools.partial(
        pl.kernel,
        mesh=mesh,
        out_type=jax.ShapeDtypeStruct((seq, D_MODEL, batch), jnp.float32),
        scratch_types=[
            pltpu.VMEM((seq, BLK), jnp.int32),
            pltpu.VMEM((2, D_MODEL, BLK), jnp.int32),
            pltpu.VMEM((2, D_MODEL, BLK), jnp.float32),
            pltpu.SemaphoreType.DMA,
            pltpu.SemaphoreType.DMA((2,)),
            pltpu.SemaphoreType.DMA((2,)),
        ],
        compiler_params=pltpu.CompilerParams(needs_layout_passes=False),
    )
    def gather_kernel(idx_hbm, flat_hbm, out_hbm,
                      idx_v, addr_v, tile_v, sem_i, sem_g, sem_w):
        wid = lax.axis_index("s") * 2 + lax.axis_index("c")
        col0 = wid * BLK

        pltpu.async_copy(
            idx_hbm.at[:, pl.ds(col0, BLK)], idx_v, sem_i
        ).wait()

        def addr_gen(s, b):
            @plsc.parallel_loop(0, D_MODEL, unroll=4)
            def body(d):
                base = d * vpad
                for q in range(BLK // 16):
                    sl = pl.ds(q * 16, 16)
                    addr_v[b, d, sl] = idx_v[s, sl] + base

        def fire_g(b):
            for d in range(D_MODEL):
                pltpu.async_copy(
                    flat_hbm.at[addr_v.at[b, d]], tile_v.at[b, d], sem_g.at[b]
                )

        def wait_g(b):
            for d in range(D_MODEL):
                pltpu.make_async_copy(
                    flat_hbm.at[addr_v.at[b, d]], tile_v.at[b, d], sem_g.at[b]
                ).wait()

        def fire_w(s, b):
            pltpu.async_copy(
                tile_v.at[b], out_hbm.at[s, :, pl.ds(col0, BLK)], sem_w.at[b]
            )

        def wait_w(b):
            pltpu.make_async_copy(
                tile_v.at[b], out_hbm.at[0, :, pl.ds(col0, BLK)], sem_w.at[b]
            ).wait()

        addr_gen(0, 0)
        fire_g(0)

        def step(j, carry):
            s0 = 2 * j
            addr_gen(s0 + 1, 1)
            wait_g(0)
            pl.when(j > 0)(lambda: wait_w(1))
            fire_g(1)
            fire_w(s0, 0)
            addr_gen(s0 + 2, 0)
            wait_g(1)
            wait_w(0)
            fire_g(0)
            fire_w(s0 + 1, 1)
            return carry

        n_main = (seq - 2) // 2
        lax.fori_loop(0, n_main, step, 0)

        s0 = 2 * n_main
        addr_gen(s0 + 1, 1)
        wait_g(0)
        if n_main > 0:
            wait_w(1)
        fire_g(1)
        fire_w(s0, 0)
        wait_g(1)
        wait_w(0)
        fire_w(s0 + 1, 1)
        wait_w(1)

    return gather_kernel


def kernel(input, table):
    b, s = input.shape
    v, d = table.shape
    vpad = (v + 127) // 128 * 128
    idx_t = input.T
    flat = jnp.pad(table.T, ((0, 0), (0, vpad - v))).reshape(-1)
    out5 = _make_gather(s, b, vpad)(idx_t, flat)
    return jnp.transpose(out5, (2, 0, 1))


# speedup vs baseline: 7.1527x; 7.1527x over previous
import functools
import jax
import jax.numpy as jnp
from jax import lax
from jax.experimental import pallas as pl
from jax.experimental.pallas import tpu as pltpu
from jax.experimental.pallas import tpu_sc as plsc

D_MODEL = 64
NUM_WORKERS = 32
BLK = 128
NBUF = 3


def _make_gather(seq: int, batch: int):
    assert batch // BLK == NUM_WORKERS
    mesh = plsc.VectorSubcoreMesh(core_axis_name="c", subcore_axis_name="s")

    @functools.partial(
        pl.kernel,
        mesh=mesh,
        out_type=jax.ShapeDtypeStruct((seq, D_MODEL, batch), jnp.float32),
        scratch_types=[
            pltpu.VMEM((seq, BLK), jnp.int32),
            pltpu.VMEM((NBUF, BLK, 2 * D_MODEL), jnp.float32),
            pltpu.VMEM((NBUF, D_MODEL, BLK), jnp.float32),
            pltpu.SemaphoreType.DMA,
            pltpu.SemaphoreType.DMA((NBUF,)),
            pltpu.SemaphoreType.DMA((NBUF,)),
        ],
        compiler_params=pltpu.CompilerParams(needs_layout_passes=False),
    )
    def gather_kernel(idx_hbm, table_hbm, out_hbm,
                      idx_v, rows_v, tile_v, sem_i, sem_g, sem_w):
        wid = lax.axis_index("s") * 2 + lax.axis_index("c")
        col0 = wid * BLK

        pltpu.async_copy(
            idx_hbm.at[:, pl.ds(col0, BLK)], idx_v, sem_i
        ).wait()

        def fire_g(s, b):
            pltpu.async_copy(
                table_hbm.at[idx_v.at[s]], rows_v.at[b], sem_g.at[b]
            )

        def wait_g(b):
            pltpu.make_async_copy(
                table_hbm.at[idx_v.at[0]], rows_v.at[b], sem_g.at[b]
            ).wait()

        def extract(b):
            rows16 = [lax.iota(jnp.int32, 16) + (q * 16) for q in range(8)]

            @plsc.parallel_loop(0, D_MODEL, unroll=4)
            def body(d):
                dv = jnp.zeros((16,), jnp.int32) + d
                for q in range(8):
                    tile_v[b, d, pl.ds(q * 16, 16)] = plsc.load_gather(
                        rows_v.at[b], [rows16[q], dv]
                    )

        def fire_w(s, b):
            pltpu.async_copy(
                tile_v.at[b], out_hbm.at[s, :, pl.ds(col0, BLK)], sem_w.at[b]
            )

        def wait_w(b):
            pltpu.make_async_copy(
                tile_v.at[b], out_hbm.at[0, :, pl.ds(col0, BLK)], sem_w.at[b]
            ).wait()

        for b in range(NBUF):
            fire_g(b, b)

        def step(j, carry):
            s0 = NBUF * j
            for b in range(NBUF):
                s = s0 + b
                wait_g(b)
                pl.when(j > 0)(lambda b=b: wait_w(b))
                extract(b)
                fire_w(s, b)
                fire_g(s + NBUF, b)
            return carry

        n_main = seq // NBUF - 1
        lax.fori_loop(0, n_main, step, 0)

        for s in range(n_main * NBUF, seq):
            b = s % NBUF
            wait_g(b)
            wait_w(b)
            extract(b)
            fire_w(s, b)
            if s + NBUF < seq:
                fire_g(s + NBUF, b)
        for b in range(NBUF):
            wait_w(b)

    return gather_kernel


def kernel(input, table):
    b, s = input.shape
    idx_t = input.T
    table_p = jnp.pad(table, ((0, 0), (0, 2 * D_MODEL - table.shape[1])))
    out5 = _make_gather(s, b)(idx_t, table_p)
    return jnp.transpose(out5, (2, 0, 1))
